# baseline (speedup 1.0000x reference)
import jax
import jax.numpy as jnp
from jax import lax
from jax.experimental import pallas as pl
from jax.experimental.pallas import tpu as pltpu
from jax.experimental.pallas import tpu_sc as plsc

_B = 16384
_DIM = 64
_LANES = 16

_info = plsc.get_sparse_core_info()
_NC, _NS = _info.num_cores, _info.num_subcores
_NW = _NC * _NS
_BPW = _B // _NW
_CH = 128
_NR = _BPW // _CH


def _body(target_hbm, context_hbm, table_hbm, out_hbm,
          raw_t, raw_c, phys_t, phys_c, half_t, half_c,
          u_bufs, v_bufs, out_v, sem):
    wid = lax.axis_index("s") * _NC + lax.axis_index("c")
    base = wid * _BPW

    pltpu.sync_copy(target_hbm.at[pl.ds(base, _BPW)], raw_t)
    pltpu.sync_copy(context_hbm.at[pl.ds(base, _BPW)], raw_c)

    for raw, phys, half in ((raw_t, phys_t, half_t), (raw_c, phys_c, half_c)):
        def split(i, _):
            v = raw[pl.ds(i * _LANES, _LANES)]
            k = i // (_CH // _LANES)
            j = lax.rem(i, _CH // _LANES)
            phys[k, pl.ds(j * _LANES, _LANES)] = v >> 1
            half[pl.ds(i * _LANES, _LANES)] = (v & 1) * _DIM
            return 0
        lax.fori_loop(0, _BPW // _LANES, split, 0)

    def fire(r):
        return (pltpu.async_copy(table_hbm.at[phys_t.at[r]],
                                 u_bufs.at[r % 2], sem),
                pltpu.async_copy(table_hbm.at[phys_c.at[r]],
                                 v_bufs.at[r % 2], sem))

    iota = lax.iota(jnp.int32, _LANES)
    inflight = fire(0)
    for r in range(_NR):
        for c in inflight:
            c.wait()
        if r + 1 < _NR:
            nxt = fire(r + 1)
        u_b, v_b = u_bufs.at[r % 2], v_bufs.at[r % 2]

        def group(g, _):
            rows = g * _LANES + iota
            col_u = half_t[pl.ds(r * _CH + g * _LANES, _LANES)]
            col_v = half_c[pl.ds(r * _CH + g * _LANES, _LANES)]
            acc = jnp.zeros((_LANES,), jnp.float32)
            for j in range(_DIM):
                ug = plsc.load_gather(u_b, [rows, col_u + j])
                vg = plsc.load_gather(v_b, [rows, col_v + j])
                acc = acc + ug * vg
            out_v[pl.ds(r * _CH + g * _LANES, _LANES)] = acc
            return 0

        lax.fori_loop(0, _CH // _LANES, group, 0)
        if r + 1 < _NR:
            inflight = nxt

    pltpu.sync_copy(out_v, out_hbm.at[pl.ds(base, _BPW)])


@jax.jit
def kernel(target, context, embedding_weights):
    table2 = embedding_weights.reshape(500000, 2 * _DIM)
    mesh = plsc.VectorSubcoreMesh(core_axis_name="c", subcore_axis_name="s")
    run = pl.kernel(
        _body,
        out_type=jax.ShapeDtypeStruct((_B,), jnp.float32),
        mesh=mesh,
        compiler_params=pltpu.CompilerParams(needs_layout_passes=False),
        scratch_types=[
            pltpu.VMEM((_BPW,), jnp.int32),
            pltpu.VMEM((_BPW,), jnp.int32),
            pltpu.VMEM((_NR, _CH), jnp.int32),
            pltpu.VMEM((_NR, _CH), jnp.int32),
            pltpu.VMEM((_BPW,), jnp.int32),
            pltpu.VMEM((_BPW,), jnp.int32),
            pltpu.VMEM((2, _CH, 2 * _DIM), jnp.float32),
            pltpu.VMEM((2, _CH, 2 * _DIM), jnp.float32),
            pltpu.VMEM((_BPW,), jnp.float32),
            pltpu.SemaphoreType.DMA,
        ],
    )
    return run(target.astype(jnp.int32), context.astype(jnp.int32), table2)
